# Initial kernel scaffold; baseline (speedup 1.0000x reference)
import jax
import jax.numpy as jnp
from jax.experimental import pallas as pl


def kernel(x, W):
    raise NotImplementedError("write your pallas kernel here")


# speedup vs baseline: 1.1583x; 1.1583x over previous
import functools

import jax
import jax.numpy as jnp
from jax import lax
from jax.experimental import pallas as pl
from jax.experimental.pallas import tpu as pltpu
from jax.experimental.pallas import tpu_sc as plsc

VOCAB = 100000
EMBED_DIM = 128
BATCH = 4096
NUM_FIELDS = 26

NC = 2
NS = 16
NW = NC * NS
TOTAL = BATCH * NUM_FIELDS
BPW = TOTAL // NW
CH = 128
NCH = BPW // CH

_MESH = plsc.VectorSubcoreMesh(
    core_axis_name="c", subcore_axis_name="s", num_cores=NC, num_subcores=NS
)


@functools.partial(
    pl.kernel,
    out_type=jax.ShapeDtypeStruct((TOTAL, EMBED_DIM), jnp.float32),
    mesh=_MESH,
    scratch_types=[
        pltpu.VMEM((NCH, CH), jnp.int32),
        pltpu.VMEM((CH, EMBED_DIM), jnp.float32),
        pltpu.SemaphoreType.DMA,
    ],
)
def _sc_gather(idx_hbm, table_hbm, out_hbm, idx_v, buf, sem):
    wid = lax.axis_index("s") * NC + lax.axis_index("c")
    base = wid * BPW
    pltpu.sync_copy(idx_hbm.at[wid], idx_v)

    @pl.loop(0, NCH)
    def _chunk(c):
        pltpu.async_copy(table_hbm.at[idx_v.at[c]], buf, sem).wait()
        pltpu.sync_copy(buf, out_hbm.at[pl.ds(base + c * CH, CH)])


def kernel(x, W):
    idx = x.astype(jnp.int32).reshape(NW, NCH, CH)
    out = _sc_gather(idx, W)
    return out.reshape(BATCH, NUM_FIELDS, EMBED_DIM)


# speedup vs baseline: 1.2811x; 1.1061x over previous
import functools

import jax
import jax.numpy as jnp
from jax import lax
from jax.experimental import pallas as pl
from jax.experimental.pallas import tpu as pltpu
from jax.experimental.pallas import tpu_sc as plsc

VOCAB = 100000
EMBED_DIM = 128
BATCH = 4096
NUM_FIELDS = 26

NC = 2
NS = 16
NW = NC * NS
TOTAL = BATCH * NUM_FIELDS
BPW = TOTAL // NW
CH = 128
NCH = BPW // CH
NBUF = 2
assert NCH % NBUF == 0

_MESH = plsc.VectorSubcoreMesh(
    core_axis_name="c", subcore_axis_name="s", num_cores=NC, num_subcores=NS
)


@functools.partial(
    pl.kernel,
    out_type=jax.ShapeDtypeStruct((TOTAL, EMBED_DIM), jnp.float32),
    mesh=_MESH,
    scratch_types=[
        pltpu.VMEM((NCH, CH), jnp.int32),
        [pltpu.VMEM((CH, EMBED_DIM), jnp.float32) for _ in range(NBUF)],
        [pltpu.SemaphoreType.DMA for _ in range(NBUF)],
        [pltpu.SemaphoreType.DMA for _ in range(NBUF)],
    ],
)
def _sc_gather(idx_hbm, table_hbm, out_hbm, idx_v, bufs, gsems, wsems):
    wid = lax.axis_index("s") * NC + lax.axis_index("c")
    base = wid * BPW
    pltpu.sync_copy(idx_hbm.at[wid], idx_v)

    for b in range(NBUF):
        pltpu.async_copy(table_hbm.at[idx_v.at[b]], bufs[b], gsems[b])

    @pl.loop(0, NCH, step=NBUF)
    def _chunk(c):
        for b in range(NBUF):
            cur = c + b
            pltpu.make_async_copy(
                table_hbm.at[idx_v.at[b]], bufs[b], gsems[b]
            ).wait()
            out_slc = out_hbm.at[pl.ds(base + cur * CH, CH)]
            pltpu.async_copy(bufs[b], out_slc, wsems[b])

            @pl.when(cur + NBUF < NCH)
            def _refill():
                pltpu.make_async_copy(bufs[b], out_slc, wsems[b]).wait()
                pltpu.async_copy(
                    table_hbm.at[idx_v.at[cur + NBUF]], bufs[b], gsems[b]
                )

    for b in range(NBUF):
        pltpu.make_async_copy(
            bufs[b], out_hbm.at[pl.ds(base, CH)], wsems[b]
        ).wait()


def kernel(x, W):
    idx = x.astype(jnp.int32).reshape(NW, NCH, CH)
    out = _sc_gather(idx, W)
    return out.reshape(BATCH, NUM_FIELDS, EMBED_DIM)


# speedup vs baseline: 1.3072x; 1.0204x over previous
import functools

import jax
import jax.numpy as jnp
from jax import lax
from jax.experimental import pallas as pl
from jax.experimental.pallas import tpu as pltpu
from jax.experimental.pallas import tpu_sc as plsc

VOCAB = 100000
EMBED_DIM = 128
BATCH = 4096
NUM_FIELDS = 26

NC = 2
NS = 16
NW = NC * NS
TOTAL = BATCH * NUM_FIELDS
BPW = TOTAL // NW
CH = 104
NCH = BPW // CH
NBUF = 8
assert NCH % NBUF == 0

_MESH = plsc.VectorSubcoreMesh(
    core_axis_name="c", subcore_axis_name="s", num_cores=NC, num_subcores=NS
)


@functools.partial(
    pl.kernel,
    out_type=jax.ShapeDtypeStruct((TOTAL, EMBED_DIM), jnp.float32),
    mesh=_MESH,
    scratch_types=[
        pltpu.VMEM((NCH, CH), jnp.int32),
        [pltpu.VMEM((CH, EMBED_DIM), jnp.float32) for _ in range(NBUF)],
        [pltpu.SemaphoreType.DMA for _ in range(NBUF)],
        [pltpu.SemaphoreType.DMA for _ in range(NBUF)],
    ],
)
def _sc_gather(idx_hbm, table_hbm, out_hbm, idx_v, bufs, gsems, wsems):
    wid = lax.axis_index("s") * NC + lax.axis_index("c")
    base = wid * BPW
    pltpu.sync_copy(idx_hbm.at[wid], idx_v)

    for b in range(NBUF):
        pltpu.async_copy(table_hbm.at[idx_v.at[b]], bufs[b], gsems[b])

    @pl.loop(0, NCH, step=NBUF)
    def _chunk(c):
        for b in range(NBUF):
            cur = c + b
            pltpu.make_async_copy(
                table_hbm.at[idx_v.at[b]], bufs[b], gsems[b]
            ).wait()
            out_slc = out_hbm.at[pl.ds(base + cur * CH, CH)]
            pltpu.async_copy(bufs[b], out_slc, wsems[b])

            @pl.when(cur + NBUF < NCH)
            def _refill():
                pltpu.make_async_copy(bufs[b], out_slc, wsems[b]).wait()
                pltpu.async_copy(
                    table_hbm.at[idx_v.at[cur + NBUF]], bufs[b], gsems[b]
                )

    for b in range(NBUF):
        pltpu.make_async_copy(
            bufs[b], out_hbm.at[pl.ds(base, CH)], wsems[b]
        ).wait()


def kernel(x, W):
    idx = x.astype(jnp.int32).reshape(NW, NCH, CH)
    out = _sc_gather(idx, W)
    return out.reshape(BATCH, NUM_FIELDS, EMBED_DIM)


# speedup vs baseline: 3.7488x; 2.8678x over previous
import functools

import jax
import jax.numpy as jnp
from jax import lax
from jax.experimental import pallas as pl
from jax.experimental.pallas import tpu as pltpu
from jax.experimental.pallas import tpu_sc as plsc

VOCAB = 100000
EMBED_DIM = 128
BATCH = 4096
NUM_FIELDS = 26

NC = 2
NS = 16
NW = NC * NS
TOTAL = BATCH * NUM_FIELDS
BPW = TOTAL // NW
CH = 104
NCH = BPW // CH
NBUF = 8
assert NCH % NBUF == 0

_MESH = plsc.VectorSubcoreMesh(
    core_axis_name="c", subcore_axis_name="s", num_cores=NC, num_subcores=NS
)


@functools.partial(
    pl.kernel,
    out_type=jax.ShapeDtypeStruct((TOTAL, EMBED_DIM), jnp.float32),
    mesh=_MESH,
    scratch_types=[
        pltpu.VMEM((NCH, CH), jnp.int32),
        [pltpu.VMEM((CH, EMBED_DIM), jnp.float32) for _ in range(NBUF)],
        [pltpu.SemaphoreType.DMA for _ in range(NBUF)],
        [pltpu.SemaphoreType.DMA for _ in range(NBUF)],
    ],
)
def _sc_gather(idx_hbm, table_hbm, out_hbm, idx_v, bufs, gsems, wsems):
    wid = lax.axis_index("s") * NC + lax.axis_index("c")
    base = wid * BPW
    pltpu.sync_copy(idx_hbm.at[wid], idx_v)

    for b in range(NBUF):
        pltpu.async_copy(table_hbm.at[idx_v.at[b]], bufs[b], gsems[b])

    @pl.loop(0, NCH, step=NBUF)
    def _chunk(c):
        for b in range(NBUF):
            cur = c + b
            pltpu.make_async_copy(
                table_hbm.at[idx_v.at[b]], bufs[b], gsems[b]
            ).wait()
            out_slc = out_hbm.at[pl.ds(base + cur * CH, CH)]
            pltpu.async_copy(bufs[b], out_slc, wsems[b])

            @pl.when(cur + NBUF < NCH)
            def _refill():
                pltpu.make_async_copy(bufs[b], out_slc, wsems[b]).wait()
                pltpu.async_copy(
                    table_hbm.at[idx_v.at[cur + NBUF]], bufs[b], gsems[b]
                )

    for b in range(NBUF):
        pltpu.make_async_copy(
            bufs[b], out_hbm.at[pl.ds(base, CH)], wsems[b]
        ).wait()


def kernel(x, W):
    idx = x.T.astype(jnp.int32).reshape(NW, NCH, CH)
    out = _sc_gather(idx, W)
    return out.reshape(NUM_FIELDS, BATCH, EMBED_DIM).transpose(1, 0, 2)


# speedup vs baseline: 3.7654x; 1.0044x over previous
import functools

import jax
import jax.numpy as jnp
from jax import lax
from jax.experimental import pallas as pl
from jax.experimental.pallas import tpu as pltpu
from jax.experimental.pallas import tpu_sc as plsc

VOCAB = 100000
EMBED_DIM = 128
BATCH = 4096
NUM_FIELDS = 26

NC = 2
NS = 16
NW = NC * NS
TOTAL = BATCH * NUM_FIELDS
BPW = TOTAL // NW
CH = 64
NCH = BPW // CH
NBUF = 13
assert NCH % NBUF == 0

_MESH = plsc.VectorSubcoreMesh(
    core_axis_name="c", subcore_axis_name="s", num_cores=NC, num_subcores=NS
)


@functools.partial(
    pl.kernel,
    out_type=jax.ShapeDtypeStruct((TOTAL, EMBED_DIM), jnp.float32),
    mesh=_MESH,
    scratch_types=[
        pltpu.VMEM((NCH, CH), jnp.int32),
        [pltpu.VMEM((CH, EMBED_DIM), jnp.float32) for _ in range(NBUF)],
        [pltpu.SemaphoreType.DMA for _ in range(NBUF)],
        [pltpu.SemaphoreType.DMA for _ in range(NBUF)],
    ],
)
def _sc_gather(idx_hbm, table_hbm, out_hbm, idx_v, bufs, gsems, wsems):
    wid = lax.axis_index("s") * NC + lax.axis_index("c")
    base = wid * BPW
    pltpu.sync_copy(idx_hbm.at[wid], idx_v)

    for b in range(NBUF):
        pltpu.async_copy(table_hbm.at[idx_v.at[b]], bufs[b], gsems[b])

    @pl.loop(0, NCH, step=NBUF)
    def _chunk(c):
        for b in range(NBUF):
            cur = c + b
            pltpu.make_async_copy(
                table_hbm.at[idx_v.at[b]], bufs[b], gsems[b]
            ).wait()
            out_slc = out_hbm.at[pl.ds(base + cur * CH, CH)]
            pltpu.async_copy(bufs[b], out_slc, wsems[b])

            @pl.when(cur + NBUF < NCH)
            def _refill():
                pltpu.make_async_copy(bufs[b], out_slc, wsems[b]).wait()
                pltpu.async_copy(
                    table_hbm.at[idx_v.at[cur + NBUF]], bufs[b], gsems[b]
                )

    for b in range(NBUF):
        pltpu.make_async_copy(
            bufs[b], out_hbm.at[pl.ds(base, CH)], wsems[b]
        ).wait()


def kernel(x, W):
    idx = x.T.astype(jnp.int32).reshape(NW, NCH, CH)
    out = _sc_gather(idx, W)
    return out.reshape(NUM_FIELDS, BATCH, EMBED_DIM).transpose(1, 0, 2)
